# baseline (speedup 1.0000x reference)
import jax
import jax.numpy as jnp
from jax import lax
from jax.experimental import pallas as pl
from jax.experimental.pallas import tpu as pltpu
from jax.experimental.pallas import tpu_sc as plsc

WORD = 1000000
D = 64
B = 16384
NNEG = 5

NC = 2
NS = 16
NW = NC * NS
BPW = B // NW
CHUNK = 32
NCHUNK = BPW // CHUNK
DV = D // 16

TC_FULL = WORD // 128
TC_BASE = TC_FULL // NW
TC_EXTRA = TC_FULL % NW
PAIR_TRIPS = (TC_BASE + 2) // 2


def _pack_body(ut_hbm, vt_hbm, tailu_hbm, tailv_hbm, u2_hbm, v2_hbm,
               ina, inb, outb, outpart, semin):
    wid = lax.axis_index("s") * NC + lax.axis_index("c")
    start = TC_BASE * wid + jnp.minimum(wid, TC_EXTRA)
    last = start + TC_BASE + jnp.where(wid < TC_EXTRA, 1, 0) - 1

    lanes = jax.lax.iota(jnp.int32, 16)
    dvecs = [q * 16 + lanes for q in range(DV)]

    def permute(src, dst, nrows, coloff=0):
        @plsc.parallel_loop(0, nrows, unroll=8)
        def row(i):
            for h in range(2):
                col = jnp.full((16,), coloff + 2 * i + h, jnp.int32)
                for q in range(DV):
                    g = plsc.load_gather(src, [dvecs[q], col])
                    dst[i, pl.ds(h * 64 + q * 16, 16)] = g

    for tab_hbm, tail_hbm, out_hbm in ((ut_hbm, tailu_hbm, u2_hbm),
                                       (vt_hbm, tailv_hbm, v2_hbm)):
        def fire(tc, buf):
            pltpu.async_copy(
                tab_hbm.at[pl.ds(0, D), pl.ds(tc * 128, 128)],
                buf.at[pl.ds(0, D), pl.ds(0, 128)], semin)

        def drain(buf):
            pltpu.make_async_copy(
                tab_hbm.at[pl.ds(0, D), pl.ds(0, 128)],
                buf.at[pl.ds(0, D), pl.ds(0, 128)], semin).wait()

        def clamp(tc):
            return jnp.minimum(tc, last)

        fire(clamp(start), ina)
        fire(clamp(start + 1), inb)

        def pair(k, _):
            tca = clamp(start + 2 * k)
            tcb = clamp(start + 2 * k + 1)
            drain(ina)
            permute(ina, outb, 64)
            fire(clamp(start + 2 * k + 2), ina)
            pltpu.sync_copy(outb, out_hbm.at[pl.ds(tca * 64, 64)])
            drain(inb)
            permute(inb, outb, 64)
            fire(clamp(start + 2 * k + 3), inb)
            pltpu.sync_copy(outb, out_hbm.at[pl.ds(tcb * 64, 64)])
            return 0

        lax.fori_loop(0, PAIR_TRIPS, pair, 0)
        drain(ina)
        drain(inb)

        @pl.when(wid == NW - 1)
        def _():
            pltpu.sync_copy(tail_hbm, outpart)
            pltpu.sync_copy(outpart, out_hbm.at[pl.ds(TC_FULL * 64, 32)])


def _sc_body(u_hbm, v_hbm,
             urow_hbm, vrow_hbm, nrow_hbm, upar_hbm, vpar_hbm, npar_hbm,
             out1_hbm, out2_hbm,
             urow, vrow, nrow, upar, vpar, npar,
             ubuf0, vbuf0, nbuf00, nbuf01, nbuf02, nbuf03, nbuf04,
             ubuf1, vbuf1, nbuf10, nbuf11, nbuf12, nbuf13, nbuf14,
             accbuf, sem0, sem1):
    wid = lax.axis_index("s") * NC + lax.axis_index("c")
    bufs = [
        (ubuf0, vbuf0, [nbuf00, nbuf01, nbuf02, nbuf03, nbuf04], sem0),
        (ubuf1, vbuf1, [nbuf10, nbuf11, nbuf12, nbuf13, nbuf14], sem1),
    ]

    pltpu.sync_copy(urow_hbm.at[wid], urow)
    pltpu.sync_copy(vrow_hbm.at[wid], vrow)
    pltpu.sync_copy(nrow_hbm.at[wid], nrow)
    pltpu.sync_copy(upar_hbm.at[wid], upar)
    pltpu.sync_copy(vpar_hbm.at[wid], vpar)
    pltpu.sync_copy(npar_hbm.at[wid], npar)

    def fire(c, p):
        ub, vb, nb, sem = bufs[p]
        pltpu.async_copy(u_hbm.at[urow.at[c]], ub, sem)
        pltpu.async_copy(v_hbm.at[vrow.at[c]], vb, sem)
        for n in range(NNEG):
            pltpu.async_copy(v_hbm.at[nrow.at[n, c]], nb[n], sem)

    def drain(p):
        ub, vb, nb, sem = bufs[p]
        pltpu.make_async_copy(u_hbm.at[pl.ds(0, CHUNK)], ub, sem).wait()
        pltpu.make_async_copy(v_hbm.at[pl.ds(0, CHUNK)], vb, sem).wait()
        for n in range(NNEG):
            pltpu.make_async_copy(v_hbm.at[pl.ds(0, CHUNK)], nb[n], sem).wait()

    def dot_pass(c, ub, pb, pbpar_slice, acc):
        def body(t, a):
            base = t * 16
            pu = upar[c, pl.ds(base, 16)]
            pp = pbpar_slice(pl.ds(base, 16))
            for l in range(16):
                r = base + l
                ou = pu[l] * 64
                op = pp[l] * 64
                for q in range(DV):
                    a = a + (ub[r, pl.ds(ou + 16 * q, 16)]
                             * pb[r, pl.ds(op + 16 * q, 16)])
            return a

        return lax.fori_loop(0, CHUNK // 16, body, acc)

    def compute(c, p, accs):
        ub, vb, nb, _ = bufs[p]
        a1, a2 = accs
        a1 = dot_pass(c, ub, vb, lambda s: vpar[c, s], a1)
        for n in range(NNEG):
            a2 = dot_pass(c, ub, nb[n], lambda s, n=n: npar[n, c, s], a2)
        return (a1, a2)

    z = jnp.zeros((16,), jnp.float32)
    fire(0, 0)
    fire(1, 1)

    def chunk_pair(g, accs):
        c0 = 2 * g
        drain(0)
        accs = compute(c0, 0, accs)

        @pl.when(c0 + 2 < NCHUNK)
        def _():
            fire(c0 + 2, 0)

        drain(1)
        accs = compute(c0 + 1, 1, accs)

        @pl.when(c0 + 3 < NCHUNK)
        def _():
            fire(c0 + 3, 1)

        return accs

    accs = lax.fori_loop(0, NCHUNK // 2, chunk_pair, (z, z))

    accbuf[...] = accs[0]
    pltpu.sync_copy(accbuf, out1_hbm.at[wid])
    accbuf[...] = accs[1]
    pltpu.sync_copy(accbuf, out2_hbm.at[wid])


@jax.jit
def _skipgram(ut, vt, tail_u, tail_v,
              urow_w, vrow_w, nrow_w, upar_w, vpar_w, npar_w):
    mesh = plsc.VectorSubcoreMesh(core_axis_name="c", subcore_axis_name="s")

    pack = pl.kernel(
        _pack_body,
        out_type=(
            jax.ShapeDtypeStruct((WORD // 2, 2 * D), jnp.float32),
            jax.ShapeDtypeStruct((WORD // 2, 2 * D), jnp.float32),
        ),
        mesh=mesh,
        compiler_params=pltpu.CompilerParams(needs_layout_passes=False),
        scratch_types=[
            pltpu.VMEM((D, 129), jnp.float32),
            pltpu.VMEM((D, 129), jnp.float32),
            pltpu.VMEM((D, 128), jnp.float32),
            pltpu.VMEM((32, 128), jnp.float32),
            pltpu.SemaphoreType.DMA,
        ],
    )
    u2, v2 = pack(ut, vt, tail_u, tail_v)

    row = pltpu.VMEM((CHUNK, 2 * D), jnp.float32)
    f = pl.kernel(
        _sc_body,
        out_type=(
            jax.ShapeDtypeStruct((NW, 16), jnp.float32),
            jax.ShapeDtypeStruct((NW, 16), jnp.float32),
        ),
        mesh=mesh,
        scratch_types=[
            pltpu.VMEM((NCHUNK, CHUNK), jnp.int32),
            pltpu.VMEM((NCHUNK, CHUNK), jnp.int32),
            pltpu.VMEM((NNEG, NCHUNK, CHUNK), jnp.int32),
            pltpu.VMEM((NCHUNK, CHUNK), jnp.int32),
            pltpu.VMEM((NCHUNK, CHUNK), jnp.int32),
            pltpu.VMEM((NNEG, NCHUNK, CHUNK), jnp.int32),
            row, row, row, row, row, row, row,
            row, row, row, row, row, row, row,
            pltpu.VMEM((16,), jnp.float32),
            pltpu.SemaphoreType.DMA,
            pltpu.SemaphoreType.DMA,
        ],
    )
    out1, out2 = f(u2, v2, urow_w, vrow_w, nrow_w, upar_w, vpar_w, npar_w)
    s1 = jnp.sum(out1)
    s2 = jnp.sum(out2)
    return -(jax.nn.log_sigmoid(s1) + jax.nn.log_sigmoid(-s2))


def kernel(u_table, v_table, pos_u, pos_v, neg_v):
    urow_w = (pos_u >> 1).reshape(NW, NCHUNK, CHUNK)
    upar_w = (pos_u & 1).reshape(NW, NCHUNK, CHUNK)
    vrow_w = (pos_v >> 1).reshape(NW, NCHUNK, CHUNK)
    vpar_w = (pos_v & 1).reshape(NW, NCHUNK, CHUNK)
    neg_t = neg_v.reshape(NW, NCHUNK, CHUNK, NNEG).transpose(0, 3, 1, 2)
    nrow_w = neg_t >> 1
    npar_w = neg_t & 1
    tail_u = u_table[WORD - 64:].reshape(32, 2 * D)
    tail_v = v_table[WORD - 64:].reshape(32, 2 * D)
    return _skipgram(u_table.T, v_table.T, tail_u, tail_v,
                     urow_w, vrow_w, nrow_w, upar_w, vpar_w, npar_w)


# speedup vs baseline: 1.6307x; 1.6307x over previous
import jax
import jax.numpy as jnp
from jax import lax
from jax.experimental import pallas as pl
from jax.experimental.pallas import tpu as pltpu
from jax.experimental.pallas import tpu_sc as plsc

WORD = 1000000
D = 64
B = 16384
NNEG = 5

NC = 2
NS = 16
NW = NC * NS
BPW = B // NW
CHUNK = 128
NCHUNK = BPW // CHUNK
DV = D // 16


def _sc_body(u_hbm, v_hbm, posu_hbm, posv_hbm, negw_hbm, out1_hbm, out2_hbm,
             uidx, vidx, nidx,
             ubuf0, vbuf0, nbuf00, nbuf01, nbuf02, nbuf03, nbuf04,
             ubuf1, vbuf1, nbuf10, nbuf11, nbuf12, nbuf13, nbuf14,
             accbuf, sem0, sem1):
    wid = lax.axis_index("s") * NC + lax.axis_index("c")
    bufs = [
        (ubuf0, vbuf0, [nbuf00, nbuf01, nbuf02, nbuf03, nbuf04], sem0),
        (ubuf1, vbuf1, [nbuf10, nbuf11, nbuf12, nbuf13, nbuf14], sem1),
    ]

    pltpu.sync_copy(posu_hbm.at[wid], uidx)
    pltpu.sync_copy(posv_hbm.at[wid], vidx)
    pltpu.sync_copy(negw_hbm.at[wid], nidx)

    def fire(c, p):
        ub, vb, nb, sem = bufs[p]
        ds = [pltpu.async_copy(u_hbm.at[uidx.at[c]], ub, sem),
              pltpu.async_copy(v_hbm.at[vidx.at[c]], vb, sem)]
        for n in range(NNEG):
            ds.append(pltpu.async_copy(v_hbm.at[nidx.at[n, c]], nb[n], sem))
        return ds

    def compute(p, accs):
        ub, vb, nb, _ = bufs[p]

        def body(r, a):
            a1, a20, a21, a22, a23, a24 = a
            for q in range(DV):
                s = pl.ds(16 * q, 16)
                u = ub[r, s]
                a1 = a1 + u * vb[r, s]
                a20 = a20 + u * nb[0][r, s]
                a21 = a21 + u * nb[1][r, s]
                a22 = a22 + u * nb[2][r, s]
                a23 = a23 + u * nb[3][r, s]
                a24 = a24 + u * nb[4][r, s]
            return (a1, a20, a21, a22, a23, a24)

        return lax.fori_loop(0, CHUNK, body, accs)

    z = jnp.zeros((16,), jnp.float32)
    accs = (z, z, z, z, z, z)
    inflight = fire(0, 0)
    for c in range(NCHUNK):
        p = c % 2
        for d in inflight:
            d.wait()
        if c + 1 < NCHUNK:
            inflight = fire(c + 1, 1 - p)
        accs = compute(p, accs)

    accbuf[...] = accs[0]
    pltpu.sync_copy(accbuf, out1_hbm.at[wid])
    accbuf[...] = accs[1] + accs[2] + accs[3] + accs[4] + accs[5]
    pltpu.sync_copy(accbuf, out2_hbm.at[wid])


@jax.jit
def _skipgram(u_table, v_table, pos_u, pos_v, neg_w):
    mesh = plsc.VectorSubcoreMesh(core_axis_name="c", subcore_axis_name="s")
    row = pltpu.VMEM((CHUNK, D), jnp.float32)
    f = pl.kernel(
        _sc_body,
        out_type=(
            jax.ShapeDtypeStruct((NW, 16), jnp.float32),
            jax.ShapeDtypeStruct((NW, 16), jnp.float32),
        ),
        mesh=mesh,
        compiler_params=pltpu.CompilerParams(use_tc_tiling_on_sc=False),
        scratch_types=[
            pltpu.VMEM((NCHUNK, CHUNK), jnp.int32),
            pltpu.VMEM((NCHUNK, CHUNK), jnp.int32),
            pltpu.VMEM((NNEG, NCHUNK, CHUNK), jnp.int32),
            row, row, row, row, row, row, row,
            row, row, row, row, row, row, row,
            pltpu.VMEM((16,), jnp.float32),
            pltpu.SemaphoreType.DMA,
            pltpu.SemaphoreType.DMA,
        ],
    )
    out1, out2 = f(u_table, v_table, pos_u, pos_v, neg_w)
    s1 = jnp.sum(out1)
    s2 = jnp.sum(out2)
    return -(jax.nn.log_sigmoid(s1) + jax.nn.log_sigmoid(-s2))


def kernel(u_table, v_table, pos_u, pos_v, neg_v):
    pos_u_w = pos_u.reshape(NW, NCHUNK, CHUNK)
    pos_v_w = pos_v.reshape(NW, NCHUNK, CHUNK)
    neg_w = neg_v.reshape(NW, NCHUNK, CHUNK, NNEG).transpose(0, 3, 1, 2)
    return _skipgram(u_table, v_table, pos_u_w, pos_v_w, neg_w)
